# baseline (speedup 1.0000x reference)
import functools

import jax
import jax.numpy as jnp
from jax import lax
from jax.experimental import pallas as pl
from jax.experimental.pallas import tpu as pltpu
from jax.experimental.pallas import tpu_sc as plsc

BATCH = 16384
EMB = 64
HID = 128
NROW = 1000000
_C = 4096
_NBLK = (NROW + _C - 1) // _C
_HALF = _C // 2
NPAIR = _NBLK * _HALF

try:
    _INFO = plsc.get_sparse_core_info()
    _NC = _INFO.num_cores
    _NS = _INFO.num_subcores
except ValueError:
    _NC, _NS = 2, 16
_NW = _NC * _NS
_BPW = BATCH // _NW
_G = 128
_NG = _BPW // _G


def _repack_body(tt_ref, out_ref):
    t = jnp.transpose(tt_ref[...])
    out_ref[...] = jnp.concatenate([t[:_HALF], t[_HALF:]], axis=1)


_repack = pl.pallas_call(
    _repack_body,
    grid=(_NBLK,),
    in_specs=[pl.BlockSpec((EMB, _C), lambda i: (0, i))],
    out_specs=pl.BlockSpec((_HALF, 2 * EMB), lambda i: (i, 0)),
    out_shape=jax.ShapeDtypeStruct((NPAIR, 2 * EMB), jnp.float32),
)


_sc_mesh = plsc.VectorSubcoreMesh(core_axis_name="c", subcore_axis_name="s")


@functools.partial(
    pl.kernel,
    mesh=_sc_mesh,
    out_type=jax.ShapeDtypeStruct((BATCH, 2 * EMB), jnp.float32),
    scratch_types=[
        pltpu.VMEM((_BPW,), jnp.int32),
        pltpu.VMEM((_BPW, 2 * EMB), jnp.float32),
        pltpu.SemaphoreType.DMA,
        pltpu.SemaphoreType.DMA,
    ],
)
def _sc_gather(ptab_hbm, idxp_hbm, out_hbm, idx_v, rows_v, sem_i, sem):
    wid = lax.axis_index("s") * _NC + lax.axis_index("c")
    base = wid * _BPW
    pltpu.async_copy(idxp_hbm.at[pl.ds(base, _BPW)], idx_v, sem_i).wait()
    copies = []
    for j in range(_NG):
        copies.append(
            pltpu.async_copy(
                ptab_hbm.at[idx_v.at[pl.ds(j * _G, _G)]],
                rows_v.at[pl.ds(j * _G, _G)],
                sem,
            )
        )
    for c in copies:
        c.wait()
    pltpu.sync_copy(rows_v, out_hbm.at[pl.ds(base, _BPW)])


_BB = 2048


def _mlp_body(x_ref, w1_ref, b1_ref, w2_ref, b2_ref, out_ref):
    h = jnp.dot(x_ref[...], w1_ref[...], preferred_element_type=jnp.float32)
    h = jnp.maximum(h + b1_ref[...], 0.0)
    o = jnp.dot(h, w2_ref[...], preferred_element_type=jnp.float32)
    out_ref[...] = o + b2_ref[...]


_mlp = pl.pallas_call(
    _mlp_body,
    grid=(BATCH // _BB,),
    in_specs=[
        pl.BlockSpec((_BB, EMB), lambda i: (i, 0)),
        pl.BlockSpec((EMB, HID), lambda i: (0, 0)),
        pl.BlockSpec((1, HID), lambda i: (0, 0)),
        pl.BlockSpec((HID, EMB), lambda i: (0, 0)),
        pl.BlockSpec((1, EMB), lambda i: (0, 0)),
    ],
    out_specs=pl.BlockSpec((_BB, EMB), lambda i: (i, 0)),
    out_shape=jax.ShapeDtypeStruct((BATCH, EMB), jnp.float32),
)


def kernel(item_id, item_emb_table, W1, b1, W2, b2):
    idx = item_id.astype(jnp.int32)
    blk = idx // _C
    q = idx % _C
    sel = q >= _HALF
    idx_pair = blk * _HALF + jnp.where(sel, q - _HALF, q)
    ptab = _repack(item_emb_table.T)
    pairs = _sc_gather(ptab, idx_pair)
    emb = jnp.where(sel[:, None], pairs[:, EMB:], pairs[:, :EMB])
    return _mlp(emb, W1, b1.reshape(1, HID), W2, b2.reshape(1, EMB))


# speedup vs baseline: 1.0039x; 1.0039x over previous
import functools

import jax
import jax.numpy as jnp
from jax import lax
from jax.experimental import pallas as pl
from jax.experimental.pallas import tpu as pltpu
from jax.experimental.pallas import tpu_sc as plsc

BATCH = 16384
EMB = 64
HID = 128
NROW = 1000000
_C = 4096
_NBLK = (NROW + _C - 1) // _C
_HALF = _C // 2
NPAIR = _NBLK * _HALF

try:
    _INFO = plsc.get_sparse_core_info()
    _NC = _INFO.num_cores
    _NS = _INFO.num_subcores
except ValueError:
    _NC, _NS = 2, 16
_NW = _NC * _NS
_BPW = BATCH // _NW
_G = 128
_NG = _BPW // _G


def _repack_body(tt_ref, out_ref):
    eye = (lax.broadcasted_iota(jnp.int32, (EMB, EMB), 0)
           == lax.broadcasted_iota(jnp.int32, (EMB, EMB), 1)).astype(jnp.float32)
    t = lax.dot_general(tt_ref[...], eye, (((0,), (0,)), ((), ())),
                        preferred_element_type=jnp.float32)
    out_ref[...] = jnp.concatenate([t[:_HALF], t[_HALF:]], axis=1)


_repack = pl.pallas_call(
    _repack_body,
    grid=(_NBLK,),
    in_specs=[pl.BlockSpec((EMB, _C), lambda i: (0, i))],
    out_specs=pl.BlockSpec((_HALF, 2 * EMB), lambda i: (i, 0)),
    out_shape=jax.ShapeDtypeStruct((NPAIR, 2 * EMB), jnp.float32),
)


_sc_mesh = plsc.VectorSubcoreMesh(core_axis_name="c", subcore_axis_name="s")


@functools.partial(
    pl.kernel,
    mesh=_sc_mesh,
    out_type=jax.ShapeDtypeStruct((BATCH, 2 * EMB), jnp.float32),
    scratch_types=[
        pltpu.VMEM((_BPW,), jnp.int32),
        pltpu.VMEM((_BPW, 2 * EMB), jnp.float32),
        pltpu.SemaphoreType.DMA,
        pltpu.SemaphoreType.DMA,
    ],
)
def _sc_gather(ptab_hbm, idxp_hbm, out_hbm, idx_v, rows_v, sem_i, sem):
    wid = lax.axis_index("s") * _NC + lax.axis_index("c")
    base = wid * _BPW
    pltpu.async_copy(idxp_hbm.at[pl.ds(base, _BPW)], idx_v, sem_i).wait()
    copies = []
    for j in range(_NG):
        copies.append(
            pltpu.async_copy(
                ptab_hbm.at[idx_v.at[pl.ds(j * _G, _G)]],
                rows_v.at[pl.ds(j * _G, _G)],
                sem,
            )
        )
    for c in copies:
        c.wait()
    pltpu.sync_copy(rows_v, out_hbm.at[pl.ds(base, _BPW)])


_BB = 2048


def _mlp_body(x_ref, w1_ref, b1_ref, w2_ref, b2_ref, out_ref):
    h = jnp.dot(x_ref[...], w1_ref[...], preferred_element_type=jnp.float32)
    h = jnp.maximum(h + b1_ref[...], 0.0)
    o = jnp.dot(h, w2_ref[...], preferred_element_type=jnp.float32)
    out_ref[...] = o + b2_ref[...]


_mlp = pl.pallas_call(
    _mlp_body,
    grid=(BATCH // _BB,),
    in_specs=[
        pl.BlockSpec((_BB, EMB), lambda i: (i, 0)),
        pl.BlockSpec((EMB, HID), lambda i: (0, 0)),
        pl.BlockSpec((1, HID), lambda i: (0, 0)),
        pl.BlockSpec((HID, EMB), lambda i: (0, 0)),
        pl.BlockSpec((1, EMB), lambda i: (0, 0)),
    ],
    out_specs=pl.BlockSpec((_BB, EMB), lambda i: (i, 0)),
    out_shape=jax.ShapeDtypeStruct((BATCH, EMB), jnp.float32),
)


def kernel(item_id, item_emb_table, W1, b1, W2, b2):
    idx = item_id.astype(jnp.int32)
    blk = idx // _C
    q = idx % _C
    sel = q >= _HALF
    idx_pair = blk * _HALF + jnp.where(sel, q - _HALF, q)
    ptab = _repack(item_emb_table.T)
    pairs = _sc_gather(ptab, idx_pair)
    emb = jnp.where(sel[:, None], pairs[:, EMB:], pairs[:, :EMB])
    return _mlp(emb, W1, b1.reshape(1, HID), W2, b2.reshape(1, EMB))


# speedup vs baseline: 1.0989x; 1.0946x over previous
import functools

import jax
import jax.numpy as jnp
from jax import lax
from jax.experimental import pallas as pl
from jax.experimental.pallas import tpu as pltpu
from jax.experimental.pallas import tpu_sc as plsc

BATCH = 16384
EMB = 64
HID = 128
NROW = 1000000
_C = 4096
_NBLK = (NROW + _C - 1) // _C
_QUAD = _C // 4
NPAIR = _NBLK * _QUAD

try:
    _INFO = plsc.get_sparse_core_info()
    _NC = _INFO.num_cores
    _NS = _INFO.num_subcores
except ValueError:
    _NC, _NS = 2, 16
_NW = _NC * _NS
_BPW = BATCH // _NW
_G = 128
_NG = _BPW // _G


def _rne16(x):
    u = lax.bitcast_convert_type(x, jnp.uint32)
    return (u + 0x7FFF + ((u >> 16) & 1)) >> 16


def _repack_body(tt_ref, out_ref):
    eye = (lax.broadcasted_iota(jnp.int32, (EMB, EMB), 0)
           == lax.broadcasted_iota(jnp.int32, (EMB, EMB), 1)).astype(jnp.float32)
    t = lax.dot_general(tt_ref[...], eye, (((0,), (0,)), ((), ())),
                        preferred_element_type=jnp.float32)
    w_lo = _rne16(t[:_QUAD]) | (_rne16(t[_QUAD:2 * _QUAD]) << 16)
    w_hi = _rne16(t[2 * _QUAD:3 * _QUAD]) | (_rne16(t[3 * _QUAD:]) << 16)
    w = jnp.concatenate([w_lo, w_hi], axis=1)
    out_ref[...] = lax.bitcast_convert_type(w, jnp.float32)


_repack = pl.pallas_call(
    _repack_body,
    grid=(_NBLK,),
    in_specs=[pl.BlockSpec((EMB, _C), lambda i: (0, i))],
    out_specs=pl.BlockSpec((_QUAD, 2 * EMB), lambda i: (i, 0)),
    out_shape=jax.ShapeDtypeStruct((NPAIR, 2 * EMB), jnp.float32),
)


_sc_mesh = plsc.VectorSubcoreMesh(core_axis_name="c", subcore_axis_name="s")


@functools.partial(
    pl.kernel,
    mesh=_sc_mesh,
    out_type=jax.ShapeDtypeStruct((BATCH, 2 * EMB), jnp.float32),
    scratch_types=[
        pltpu.VMEM((_BPW,), jnp.int32),
        pltpu.VMEM((_BPW, 2 * EMB), jnp.float32),
        pltpu.SemaphoreType.DMA,
        pltpu.SemaphoreType.DMA,
    ],
)
def _sc_gather(ptab_hbm, idxp_hbm, out_hbm, idx_v, rows_v, sem_i, sem):
    wid = lax.axis_index("s") * _NC + lax.axis_index("c")
    base = wid * _BPW
    pltpu.async_copy(idxp_hbm.at[pl.ds(base, _BPW)], idx_v, sem_i).wait()
    copies = []
    for j in range(_NG):
        copies.append(
            pltpu.async_copy(
                ptab_hbm.at[idx_v.at[pl.ds(j * _G, _G)]],
                rows_v.at[pl.ds(j * _G, _G)],
                sem,
            )
        )
    for c in copies:
        c.wait()
    pltpu.sync_copy(rows_v, out_hbm.at[pl.ds(base, _BPW)])


_BB = 2048


def _mlp_body(x_ref, w1_ref, b1_ref, w2_ref, b2_ref, out_ref):
    h = jnp.dot(x_ref[...], w1_ref[...], preferred_element_type=jnp.float32)
    h = jnp.maximum(h + b1_ref[...], 0.0)
    o = jnp.dot(h, w2_ref[...], preferred_element_type=jnp.float32)
    out_ref[...] = o + b2_ref[...]


_mlp = pl.pallas_call(
    _mlp_body,
    grid=(BATCH // _BB,),
    in_specs=[
        pl.BlockSpec((_BB, EMB), lambda i: (i, 0)),
        pl.BlockSpec((EMB, HID), lambda i: (0, 0)),
        pl.BlockSpec((1, HID), lambda i: (0, 0)),
        pl.BlockSpec((HID, EMB), lambda i: (0, 0)),
        pl.BlockSpec((1, EMB), lambda i: (0, 0)),
    ],
    out_specs=pl.BlockSpec((_BB, EMB), lambda i: (i, 0)),
    out_shape=jax.ShapeDtypeStruct((BATCH, EMB), jnp.float32),
)


def kernel(item_id, item_emb_table, W1, b1, W2, b2):
    idx = item_id.astype(jnp.int32)
    blk = idx // _C
    q = idx % _C
    qtr = q // _QUAD
    idx_quad = blk * _QUAD + q % _QUAD
    quads = _sc_gather(_repack(item_emb_table.T), idx_quad)
    u = lax.bitcast_convert_type(quads, jnp.uint32)
    half = jnp.where((qtr >= 2)[:, None], u[:, EMB:], u[:, :EMB])
    bits = jnp.where(((qtr & 1) == 1)[:, None], half >> 16, half & 0xFFFF)
    emb = lax.bitcast_convert_type(bits << 16, jnp.float32)
    return _mlp(emb, W1, b1.reshape(1, HID), W2, b2.reshape(1, EMB))


# speedup vs baseline: 1.3918x; 1.2665x over previous
import functools

import jax
import jax.numpy as jnp
from jax import lax
from jax.experimental import pallas as pl
from jax.experimental.pallas import tpu as pltpu
from jax.experimental.pallas import tpu_sc as plsc

BATCH = 16384
EMB = 64
HID = 128
NROW = 1000000
_C = 8192
_NBLK = (NROW + _C - 1) // _C
_QUAD = _C // 4
NPAIR = _NBLK * _QUAD

try:
    _INFO = plsc.get_sparse_core_info()
    _NC = _INFO.num_cores
    _NS = _INFO.num_subcores
except ValueError:
    _NC, _NS = 2, 16
_NW = _NC * _NS
_BPW = BATCH // _NW
_G = 128
_NG = _BPW // _G


def _rne16(x):
    u = lax.bitcast_convert_type(x, jnp.uint32)
    return (u + 0x7FFF + ((u >> 16) & 1)) >> 16


def _repack_body(tt_ref, out_ref):
    eye = (lax.broadcasted_iota(jnp.int32, (EMB, EMB), 0)
           == lax.broadcasted_iota(jnp.int32, (EMB, EMB), 1)).astype(jnp.float32)
    t = lax.dot_general(tt_ref[...], eye, (((0,), (0,)), ((), ())),
                        preferred_element_type=jnp.float32)
    w_lo = _rne16(t[:_QUAD]) | (_rne16(t[_QUAD:2 * _QUAD]) << 16)
    w_hi = _rne16(t[2 * _QUAD:3 * _QUAD]) | (_rne16(t[3 * _QUAD:]) << 16)
    w = jnp.concatenate([w_lo, w_hi], axis=1)
    out_ref[...] = lax.bitcast_convert_type(w, jnp.float32)


_repack = pl.pallas_call(
    _repack_body,
    grid=(_NBLK,),
    in_specs=[pl.BlockSpec((EMB, _C), lambda i: (0, i))],
    out_specs=pl.BlockSpec((_QUAD, 2 * EMB), lambda i: (i, 0)),
    out_shape=jax.ShapeDtypeStruct((NPAIR, 2 * EMB), jnp.float32),
)


_sc_mesh = plsc.VectorSubcoreMesh(core_axis_name="c", subcore_axis_name="s")


@functools.partial(
    pl.kernel,
    mesh=_sc_mesh,
    out_type=jax.ShapeDtypeStruct((BATCH, 2 * EMB), jnp.float32),
    scratch_types=[
        pltpu.VMEM((_BPW,), jnp.int32),
        pltpu.VMEM((_BPW, 2 * EMB), jnp.float32),
        pltpu.SemaphoreType.DMA,
        pltpu.SemaphoreType.DMA,
    ],
)
def _sc_gather(ptab_hbm, idxp_hbm, out_hbm, idx_v, rows_v, sem_i, sem):
    wid = lax.axis_index("s") * _NC + lax.axis_index("c")
    base = wid * _BPW
    pltpu.async_copy(idxp_hbm.at[pl.ds(base, _BPW)], idx_v, sem_i).wait()
    copies = []
    for j in range(_NG):
        copies.append(
            pltpu.async_copy(
                ptab_hbm.at[idx_v.at[pl.ds(j * _G, _G)]],
                rows_v.at[pl.ds(j * _G, _G)],
                sem,
            )
        )
    for c in copies:
        c.wait()
    pltpu.sync_copy(rows_v, out_hbm.at[pl.ds(base, _BPW)])


_BB = 2048


def _mlp_body(x_ref, qtr_ref, w1_ref, b1_ref, w2_ref, b2_ref, out_ref):
    u = lax.bitcast_convert_type(x_ref[...], jnp.uint32)
    qtr = qtr_ref[...]
    half = jnp.where(qtr >= 2, u[:, EMB:], u[:, :EMB])
    bits = jnp.where((qtr & 1) == 1, half >> 16, half & 0xFFFF)
    x = lax.bitcast_convert_type(bits << 16, jnp.float32)
    h = jnp.dot(x, w1_ref[...], preferred_element_type=jnp.float32)
    h = jnp.maximum(h + b1_ref[...], 0.0)
    o = jnp.dot(h, w2_ref[...], preferred_element_type=jnp.float32)
    out_ref[...] = o + b2_ref[...]


_mlp = pl.pallas_call(
    _mlp_body,
    grid=(BATCH // _BB,),
    in_specs=[
        pl.BlockSpec((_BB, 2 * EMB), lambda i: (i, 0)),
        pl.BlockSpec((_BB, 1), lambda i: (i, 0)),
        pl.BlockSpec((EMB, HID), lambda i: (0, 0)),
        pl.BlockSpec((1, HID), lambda i: (0, 0)),
        pl.BlockSpec((HID, EMB), lambda i: (0, 0)),
        pl.BlockSpec((1, EMB), lambda i: (0, 0)),
    ],
    out_specs=pl.BlockSpec((_BB, EMB), lambda i: (i, 0)),
    out_shape=jax.ShapeDtypeStruct((BATCH, EMB), jnp.float32),
)


def kernel(item_id, item_emb_table, W1, b1, W2, b2):
    idx = item_id.astype(jnp.int32)
    blk = idx // _C
    q = idx % _C
    qtr = q // _QUAD
    idx_quad = blk * _QUAD + q % _QUAD
    quads = _sc_gather(_repack(item_emb_table.T), idx_quad)
    return _mlp(quads, qtr.reshape(BATCH, 1), W1,
                b1.reshape(1, HID), W2, b2.reshape(1, EMB))


# speedup vs baseline: 1.5953x; 1.1462x over previous
import functools

import jax
import jax.numpy as jnp
from jax import lax
from jax.experimental import pallas as pl
from jax.experimental.pallas import tpu as pltpu
from jax.experimental.pallas import tpu_sc as plsc

BATCH = 16384
EMB = 64
HID = 128
NROW = 1000000
_C = 16384
_NBLK = (NROW + _C - 1) // _C
_QUAD = _C // 4
NPAIR = _NBLK * _QUAD

try:
    _INFO = plsc.get_sparse_core_info()
    _NC = _INFO.num_cores
    _NS = _INFO.num_subcores
except ValueError:
    _NC, _NS = 2, 16
_NW = _NC * _NS
_BPW = BATCH // _NW
_G = 128
_NG = _BPW // _G


def _rne16(x):
    u = lax.bitcast_convert_type(x, jnp.uint32)
    return (u + 0x7FFF + ((u >> 16) & 1)) >> 16


def _repack_body(tt_ref, out_ref):
    eye = (lax.broadcasted_iota(jnp.int32, (EMB, EMB), 0)
           == lax.broadcasted_iota(jnp.int32, (EMB, EMB), 1)).astype(jnp.float32)
    t = lax.dot_general(tt_ref[...], eye, (((0,), (0,)), ((), ())),
                        preferred_element_type=jnp.float32)
    w_lo = _rne16(t[:_QUAD]) | (_rne16(t[_QUAD:2 * _QUAD]) << 16)
    w_hi = _rne16(t[2 * _QUAD:3 * _QUAD]) | (_rne16(t[3 * _QUAD:]) << 16)
    w = jnp.concatenate([w_lo, w_hi], axis=1)
    out_ref[...] = lax.bitcast_convert_type(w, jnp.float32)


_repack = pl.pallas_call(
    _repack_body,
    grid=(_NBLK,),
    in_specs=[pl.BlockSpec((EMB, _C), lambda i: (0, i))],
    out_specs=pl.BlockSpec((_QUAD, 2 * EMB), lambda i: (i, 0)),
    out_shape=jax.ShapeDtypeStruct((NPAIR, 2 * EMB), jnp.float32),
)


_sc_mesh = plsc.VectorSubcoreMesh(core_axis_name="c", subcore_axis_name="s")


@functools.partial(
    pl.kernel,
    mesh=_sc_mesh,
    out_type=jax.ShapeDtypeStruct((BATCH, 2 * EMB), jnp.float32),
    scratch_types=[
        pltpu.VMEM((_BPW,), jnp.int32),
        pltpu.VMEM((_BPW, 2 * EMB), jnp.float32),
        pltpu.SemaphoreType.DMA,
        pltpu.SemaphoreType.DMA,
    ],
)
def _sc_gather(ptab_hbm, idxp_hbm, out_hbm, idx_v, rows_v, sem_i, sem):
    wid = lax.axis_index("s") * _NC + lax.axis_index("c")
    base = wid * _BPW
    pltpu.async_copy(idxp_hbm.at[pl.ds(base, _BPW)], idx_v, sem_i).wait()
    copies = []
    for j in range(_NG):
        copies.append(
            pltpu.async_copy(
                ptab_hbm.at[idx_v.at[pl.ds(j * _G, _G)]],
                rows_v.at[pl.ds(j * _G, _G)],
                sem,
            )
        )
    for c in copies:
        c.wait()
    pltpu.sync_copy(rows_v, out_hbm.at[pl.ds(base, _BPW)])


_BB = 4096


def _mlp_body(x_ref, qtr_ref, w1_ref, b1_ref, w2_ref, b2_ref, out_ref):
    u = lax.bitcast_convert_type(x_ref[...], jnp.uint32)
    qtr = qtr_ref[...]
    half = jnp.where(qtr >= 2, u[:, EMB:], u[:, :EMB])
    bits = jnp.where((qtr & 1) == 1, half >> 16, half & 0xFFFF)
    x = lax.bitcast_convert_type(bits << 16, jnp.float32)
    h = jnp.dot(x, w1_ref[...], preferred_element_type=jnp.float32)
    h = jnp.maximum(h + b1_ref[...], 0.0)
    o = jnp.dot(h, w2_ref[...], preferred_element_type=jnp.float32)
    out_ref[...] = o + b2_ref[...]


_mlp = pl.pallas_call(
    _mlp_body,
    grid=(BATCH // _BB,),
    in_specs=[
        pl.BlockSpec((_BB, 2 * EMB), lambda i: (i, 0)),
        pl.BlockSpec((_BB, 1), lambda i: (i, 0)),
        pl.BlockSpec((EMB, HID), lambda i: (0, 0)),
        pl.BlockSpec((1, HID), lambda i: (0, 0)),
        pl.BlockSpec((HID, EMB), lambda i: (0, 0)),
        pl.BlockSpec((1, EMB), lambda i: (0, 0)),
    ],
    out_specs=pl.BlockSpec((_BB, EMB), lambda i: (i, 0)),
    out_shape=jax.ShapeDtypeStruct((BATCH, EMB), jnp.float32),
)


def kernel(item_id, item_emb_table, W1, b1, W2, b2):
    idx = item_id.astype(jnp.int32)
    blk = idx // _C
    q = idx % _C
    qtr = q // _QUAD
    idx_quad = blk * _QUAD + q % _QUAD
    quads = _sc_gather(_repack(item_emb_table.T), idx_quad)
    return _mlp(quads, qtr.reshape(BATCH, 1), W1,
                b1.reshape(1, HID), W2, b2.reshape(1, EMB))


# speedup vs baseline: 1.6456x; 1.0315x over previous
import functools

import jax
import jax.numpy as jnp
from jax import lax
from jax.experimental import pallas as pl
from jax.experimental.pallas import tpu as pltpu
from jax.experimental.pallas import tpu_sc as plsc

BATCH = 16384
EMB = 64
HID = 128
NROW = 1000000
_C = 32768
_NBLK = (NROW + _C - 1) // _C
_QUAD = _C // 4
NPAIR = _NBLK * _QUAD

try:
    _INFO = plsc.get_sparse_core_info()
    _NC = _INFO.num_cores
    _NS = _INFO.num_subcores
except ValueError:
    _NC, _NS = 2, 16
_NW = _NC * _NS
_BPW = BATCH // _NW
_G = 128
_NG = _BPW // _G


def _rne16(x):
    u = lax.bitcast_convert_type(x, jnp.uint32)
    return (u + 0x7FFF + ((u >> 16) & 1)) >> 16


def _repack_body(tt_ref, out_ref):
    eye = (lax.broadcasted_iota(jnp.int32, (EMB, EMB), 0)
           == lax.broadcasted_iota(jnp.int32, (EMB, EMB), 1)).astype(jnp.float32)
    t = lax.dot_general(tt_ref[...], eye, (((0,), (0,)), ((), ())),
                        preferred_element_type=jnp.float32)
    w_lo = _rne16(t[:_QUAD]) | (_rne16(t[_QUAD:2 * _QUAD]) << 16)
    w_hi = _rne16(t[2 * _QUAD:3 * _QUAD]) | (_rne16(t[3 * _QUAD:]) << 16)
    w = jnp.concatenate([w_lo, w_hi], axis=1)
    out_ref[...] = lax.bitcast_convert_type(w, jnp.float32)


_repack = pl.pallas_call(
    _repack_body,
    grid=(_NBLK,),
    in_specs=[pl.BlockSpec((EMB, _C), lambda i: (0, i))],
    out_specs=pl.BlockSpec((_QUAD, 2 * EMB), lambda i: (i, 0)),
    out_shape=jax.ShapeDtypeStruct((NPAIR, 2 * EMB), jnp.float32),
)


_sc_mesh = plsc.VectorSubcoreMesh(core_axis_name="c", subcore_axis_name="s")


@functools.partial(
    pl.kernel,
    mesh=_sc_mesh,
    out_type=jax.ShapeDtypeStruct((BATCH, 2 * EMB), jnp.float32),
    scratch_types=[
        pltpu.VMEM((_BPW,), jnp.int32),
        pltpu.VMEM((_BPW, 2 * EMB), jnp.float32),
        pltpu.SemaphoreType.DMA,
        pltpu.SemaphoreType.DMA,
    ],
)
def _sc_gather(ptab_hbm, idxp_hbm, out_hbm, idx_v, rows_v, sem_i, sem):
    wid = lax.axis_index("s") * _NC + lax.axis_index("c")
    base = wid * _BPW
    pltpu.async_copy(idxp_hbm.at[pl.ds(base, _BPW)], idx_v, sem_i).wait()
    copies = []
    for j in range(_NG):
        copies.append(
            pltpu.async_copy(
                ptab_hbm.at[idx_v.at[pl.ds(j * _G, _G)]],
                rows_v.at[pl.ds(j * _G, _G)],
                sem,
            )
        )
    for c in copies:
        c.wait()
    pltpu.sync_copy(rows_v, out_hbm.at[pl.ds(base, _BPW)])


_BB = 8192


def _mlp_body(x_ref, qtr_ref, w1_ref, b1_ref, w2_ref, b2_ref, out_ref):
    u = lax.bitcast_convert_type(x_ref[...], jnp.uint32)
    qtr = qtr_ref[...]
    half = jnp.where(qtr >= 2, u[:, EMB:], u[:, :EMB])
    bits = jnp.where((qtr & 1) == 1, half >> 16, half & 0xFFFF)
    x = lax.bitcast_convert_type(bits << 16, jnp.float32)
    h = jnp.dot(x, w1_ref[...], preferred_element_type=jnp.float32)
    h = jnp.maximum(h + b1_ref[...], 0.0)
    o = jnp.dot(h, w2_ref[...], preferred_element_type=jnp.float32)
    out_ref[...] = o + b2_ref[...]


_mlp = pl.pallas_call(
    _mlp_body,
    grid=(BATCH // _BB,),
    in_specs=[
        pl.BlockSpec((_BB, 2 * EMB), lambda i: (i, 0)),
        pl.BlockSpec((_BB, 1), lambda i: (i, 0)),
        pl.BlockSpec((EMB, HID), lambda i: (0, 0)),
        pl.BlockSpec((1, HID), lambda i: (0, 0)),
        pl.BlockSpec((HID, EMB), lambda i: (0, 0)),
        pl.BlockSpec((1, EMB), lambda i: (0, 0)),
    ],
    out_specs=pl.BlockSpec((_BB, EMB), lambda i: (i, 0)),
    out_shape=jax.ShapeDtypeStruct((BATCH, EMB), jnp.float32),
)


def kernel(item_id, item_emb_table, W1, b1, W2, b2):
    idx = item_id.astype(jnp.int32)
    blk = idx // _C
    q = idx % _C
    qtr = q // _QUAD
    idx_quad = blk * _QUAD + q % _QUAD
    quads = _sc_gather(_repack(item_emb_table.T), idx_quad)
    return _mlp(quads, qtr.reshape(BATCH, 1), W1,
                b1.reshape(1, HID), W2, b2.reshape(1, EMB))
